# baseline (speedup 1.0000x reference)
import functools
import math

import jax
import jax.numpy as jnp
from jax import lax
from jax.experimental import pallas as pl
from jax.experimental.pallas import tpu as pltpu
from jax.experimental.pallas import tpu_sc as plsc

N_NODES = 10000
FEAT = 128
EMBED = 64
CONTRA = 64
B = 4096
DEG = 32
P = 500
SAMPLE = math.ceil(DEG * 0.5)

NW = 32
BPW = B // NW
TP_PAD = 512
TPW = TP_PAD // NW


def _aux_body(f_ref, r_ref, pe_ref, aux_ref):
    x = f_ref[...]
    s = jnp.maximum(
        jnp.dot(x, r_ref[0:FEAT, :], preferred_element_type=jnp.float32), 0.0)
    nf = jnp.sqrt(jnp.sum(x * x, axis=1, keepdims=True))
    sn = jnp.sqrt(jnp.sum(s * s, axis=1, keepdims=True))
    pe = pe_ref[...]
    pn = jnp.sqrt(jnp.sum(pe * pe, axis=1))
    num = lax.dot_general(s, pe, (((1,), (1,)), ((), ())),
                          preferred_element_type=jnp.float32)
    ps = num / (pn[None, :] * sn + 1e-8)
    aux_ref[...] = jnp.concatenate(
        [nf, ps, jnp.zeros((x.shape[0], 12), jnp.float32)], axis=1)


def _make_aux(features, rsimTrans, pos_embs):
    rows = 2000
    return pl.pallas_call(
        _aux_body,
        grid=(N_NODES // rows,),
        in_specs=[
            pl.BlockSpec((rows, FEAT), lambda i: (i, 0)),
            pl.BlockSpec(rsimTrans.shape, lambda i: (0, 0)),
            pl.BlockSpec((3, CONTRA), lambda i: (0, 0)),
        ],
        out_specs=pl.BlockSpec((rows, 16), lambda i: (i, 0)),
        out_shape=jax.ShapeDtypeStruct((N_NODES, 16), jnp.float32),
    )(features, rsimTrans, pos_embs)


def _sc_body(features, aux, nodes, neigh, tp,
             self_out, agg_out, lsum_out, pos_out,
             nodes_v, nidx_v, tp_v, selfsp, selfaux, gbuf, gaux,
             aggbuf, lsumbuf, pgbuf, posbuf, sem, sem2):
    cid = lax.axis_index("c")
    sid = lax.axis_index("s")
    wid = sid * 2 + cid
    base = wid * BPW
    iota16 = lax.iota(jnp.int32, 16)

    pltpu.sync_copy(nodes.at[pl.ds(base, BPW)], nodes_v)
    for r in range(3):
        pltpu.sync_copy(neigh.at[r, pl.ds(base // 4, BPW // 4)], nidx_v.at[r])
    pltpu.sync_copy(tp.at[pl.ds(wid * TPW, TPW)], tp_v)
    pltpu.async_copy(features.at[nodes_v], selfsp, sem).wait()
    pltpu.async_copy(aux.at[nodes_v], selfaux, sem).wait()
    pltpu.async_copy(aux.at[tp_v], pgbuf, sem).wait()

    nvalid = jnp.clip(P - wid * TPW, 0, TPW)
    vmask = iota16 < nvalid
    posvec = jnp.zeros((16,), jnp.float32)
    for r in range(3):
        col = jnp.full((16,), 1 + r, jnp.int32)
        pvals = plsc.load_gather(pgbuf, [iota16, col])
        dd = jnp.where(vmask, 1.0 - pvals, 0.0)
        posvec = jnp.where(iota16 == r, jnp.sum(dd * dd), posvec)
    posbuf[...] = posvec
    pltpu.sync_copy(posbuf, pos_out.at[wid])

    zcol = jnp.zeros((16,), jnp.int32)

    def idx_of(i, r):
        return nidx_v.at[r, lax.div(i, 4), pl.ds(32 * lax.rem(i, 4), 32)]

    def fire(i, slot):
        for r in range(3):
            idxr = idx_of(i, r)
            pltpu.async_copy(features.at[idxr], gbuf.at[slot, r], sem)
            pltpu.async_copy(aux.at[idxr], gaux.at[slot, r], sem2)

    fire(0, 0)

    def one_center(i, slot, last):
        for r in range(3):
            idxr = idx_of(i, r)
            pltpu.make_async_copy(features.at[idxr], gbuf.at[slot, r],
                                  sem).wait()
            pltpu.make_async_copy(aux.at[idxr], gaux.at[slot, r],
                                  sem2).wait()

        if last:
            @pl.when(i + 1 < BPW)
            def _():
                fire(i + 1, 1 - slot)
        else:
            fire(i + 1, 1 - slot)

        sv = [selfsp[i, pl.ds(16 * j, 16)] for j in range(8)]
        auxrow = selfaux[i, pl.ds(0, 16)]
        nf_self = auxrow[0]
        slotcol = jnp.full((16,), slot, jnp.int32)

        def rel_body(r, lrow):
            dv0 = jnp.zeros((16,), jnp.float32)
            dv1 = jnp.zeros((16,), jnp.float32)
            for k in range(DEG):
                acc = gbuf[slot, r, k, pl.ds(0, 16)] * sv[0]
                for j in range(1, 8):
                    acc = acc + gbuf[slot, r, k, pl.ds(16 * j, 16)] * sv[j]
                tot = jnp.sum(acc)
                if k < 16:
                    dv0 = jnp.where(iota16 == k, tot, dv0)
                else:
                    dv1 = jnp.where(iota16 == (k - 16), tot, dv1)
            rcol = jnp.full((16,), 0, jnp.int32) + r
            nfv0 = plsc.load_gather(gaux, [slotcol, rcol, iota16, zcol])
            nfv1 = plsc.load_gather(gaux, [slotcol, rcol, iota16 + 16, zcol])
            colr = jnp.full((16,), 1, jnp.int32) + r
            s0 = dv0 / (nf_self * nfv0 + 1e-8)
            s1 = dv1 / (nf_self * nfv1 + 1e-8)
            sA, iA = plsc.sort_key_val(s0, iota16)
            sB, iB = plsc.sort_key_val(s1, iota16 + 16)
            rB = lax.rev(sB, (0,))
            riB = lax.rev(iB, (0,))
            ids = jnp.where(sA >= rB, iA, riB)
            cs = jnp.sum(jnp.where(iota16 == 1 + r, auxrow, 0.0))
            pssel = plsc.load_gather(gaux, [slotcol, rcol, ids, colr])
            e = pssel - cs
            lrow = jnp.where(iota16 == r, jnp.sum(e * e), lrow)
            arow = jnp.full((16,), 0, jnp.int32) + (r * BPW + i)
            for j in range(8):
                chunk = jnp.zeros((16,), jnp.float32)
                for c in range(16):
                    colv = plsc.load_gather(
                        gbuf, [slotcol, rcol, ids,
                               jnp.full((16,), 16 * j + c, jnp.int32)])
                    chunk = jnp.where(iota16 == c, jnp.sum(colv), chunk)
                plsc.store_scatter(aggbuf, [arow, iota16 + 16 * j],
                                   chunk * (1.0 / SAMPLE))
            return lrow

        lrow = lax.fori_loop(0, 3, rel_body, jnp.zeros((16,), jnp.float32))
        lsumbuf[i] = lrow

    def pair_body(p, carry):
        one_center(2 * p, 0, last=False)
        one_center(2 * p + 1, 1, last=True)
        return carry

    lax.fori_loop(0, BPW // 2, pair_body, 0)

    pltpu.sync_copy(selfsp, self_out.at[pl.ds(base, BPW)])
    for r in range(3):
        pltpu.sync_copy(aggbuf.at[pl.ds(r * BPW, BPW)],
                        agg_out.at[r, pl.ds(base, BPW)])
    pltpu.sync_copy(lsumbuf, lsum_out.at[pl.ds(base, BPW)])


_sc_gather_agg = functools.partial(
    pl.kernel,
    out_type=[
        jax.ShapeDtypeStruct((B, FEAT), jnp.float32),
        jax.ShapeDtypeStruct((3, B, FEAT), jnp.float32),
        jax.ShapeDtypeStruct((B, 16), jnp.float32),
        jax.ShapeDtypeStruct((NW, 16), jnp.float32),
    ],
    mesh=plsc.VectorSubcoreMesh(core_axis_name="c", subcore_axis_name="s",
                                num_cores=2, num_subcores=16),
    compiler_params=pltpu.CompilerParams(needs_layout_passes=False,
                                         use_tc_tiling_on_sc=False),
    scratch_types=[
        pltpu.VMEM((BPW,), jnp.int32),
        pltpu.VMEM((3, BPW // 4, FEAT), jnp.int32),
        pltpu.VMEM((TPW,), jnp.int32),
        pltpu.VMEM((BPW, FEAT), jnp.float32),
        pltpu.VMEM((BPW, 16), jnp.float32),
        pltpu.VMEM((2, 3, DEG, FEAT), jnp.float32),
        pltpu.VMEM((2, 3, DEG, 16), jnp.float32),
        pltpu.VMEM((3 * BPW, FEAT), jnp.float32),
        pltpu.VMEM((BPW, 16), jnp.float32),
        pltpu.VMEM((TPW, 16), jnp.float32),
        pltpu.VMEM((16,), jnp.float32),
        pltpu.SemaphoreType.DMA,
        pltpu.SemaphoreType.DMA,
    ],
)(_sc_body)


def _comb_body(self_ref, a1_ref, a2_ref, a3_ref, w1_ref, w2_ref, w3_ref,
               wt_ref, out_ref):
    sf = self_ref[...]
    r1 = jnp.maximum(jnp.dot(a1_ref[0], w1_ref[...],
                             preferred_element_type=jnp.float32), 0.0)
    r2 = jnp.maximum(jnp.dot(a2_ref[0], w2_ref[...],
                             preferred_element_type=jnp.float32), 0.0)
    r3 = jnp.maximum(jnp.dot(a3_ref[0], w3_ref[...],
                             preferred_element_type=jnp.float32), 0.0)
    cat = jnp.concatenate([sf, r1, r2, r3], axis=1)
    out = lax.dot_general(wt_ref[...], cat, (((0,), (1,)), ((), ())),
                          preferred_element_type=jnp.float32)
    out_ref[...] = jnp.maximum(out, 0.0)


def _combine(selfb, agg, W1, W2, W3, weight):
    blk = 1024
    aspec = lambda r: pl.BlockSpec((1, blk, FEAT), lambda i, _r=r: (_r, i, 0))
    return pl.pallas_call(
        _comb_body,
        grid=(B // blk,),
        in_specs=[
            pl.BlockSpec((blk, FEAT), lambda i: (i, 0)),
            aspec(0), aspec(1), aspec(2),
            pl.BlockSpec((FEAT, EMBED), lambda i: (0, 0)),
            pl.BlockSpec((FEAT, EMBED), lambda i: (0, 0)),
            pl.BlockSpec((FEAT, EMBED), lambda i: (0, 0)),
            pl.BlockSpec((FEAT + 3 * EMBED, EMBED), lambda i: (0, 0)),
        ],
        out_specs=pl.BlockSpec((EMBED, blk), lambda i: (0, i)),
        out_shape=jax.ShapeDtypeStruct((EMBED, B), jnp.float32),
    )(selfb, agg, agg, agg, W1, W2, W3, weight)


def _loss_body(lsum_ref, pos_ref, out_ref):
    l = lsum_ref[...]
    p = pos_ref[...]
    mask = lax.broadcasted_iota(jnp.int32, (1, 16), 1) < 3
    lt = jnp.sum(jnp.where(mask, l, 0.0))
    pt = jnp.sum(jnp.where(mask, p, 0.0))
    out_ref[...] = jnp.full((1, 1), 0.0) + (
        lt / (B * SAMPLE * 3) + pt / (P * 3))


def _loss(lsum, posp):
    return pl.pallas_call(
        _loss_body,
        grid=(1,),
        in_specs=[
            pl.BlockSpec((B, 16), lambda i: (0, 0)),
            pl.BlockSpec((NW, 16), lambda i: (0, 0)),
        ],
        out_specs=pl.BlockSpec((1, 1), lambda i: (0, 0)),
        out_shape=jax.ShapeDtypeStruct((1, 1), jnp.float32),
    )(lsum, posp)


def kernel(nodes, labels, neigh1, neigh2, neigh3, train_pos, features,
           weight, rsimTrans, pos_embs, W1, W2, W3):
    nodes = nodes.astype(jnp.int32)
    neigh = jnp.stack([neigh1, neigh2, neigh3]).astype(jnp.int32)
    neigh = neigh.reshape(3, B // 4, 4 * DEG)
    tp = jnp.pad(train_pos.astype(jnp.int32), (0, TP_PAD - P))
    aux = _make_aux(features, rsimTrans, pos_embs)
    selfb, agg, lsum, posp = _sc_gather_agg(features, aux, nodes, neigh, tp)
    combined = _combine(selfb, agg, W1, W2, W3, weight)
    loss = _loss(lsum, posp)
    return combined, loss.reshape(())


# speedup vs baseline: 2.0655x; 2.0655x over previous
import functools
import math

import jax
import jax.numpy as jnp
from jax import lax
from jax.experimental import pallas as pl
from jax.experimental.pallas import tpu as pltpu
from jax.experimental.pallas import tpu_sc as plsc

N_NODES = 10000
FEAT = 128
EMBED = 64
CONTRA = 64
B = 4096
DEG = 32
P = 500
SAMPLE = math.ceil(DEG * 0.5)

NW = 32
BPW = B // NW
TP_PAD = 512
TPW = TP_PAD // NW


def _aux_body(f_ref, r_ref, pe_ref, aux_ref):
    x = f_ref[...]
    s = jnp.maximum(
        jnp.dot(x, r_ref[0:FEAT, :], preferred_element_type=jnp.float32), 0.0)
    nf = jnp.sqrt(jnp.sum(x * x, axis=1, keepdims=True))
    sn = jnp.sqrt(jnp.sum(s * s, axis=1, keepdims=True))
    pe = pe_ref[...]
    pn = jnp.sqrt(jnp.sum(pe * pe, axis=1))
    num = lax.dot_general(s, pe, (((1,), (1,)), ((), ())),
                          preferred_element_type=jnp.float32)
    ps = num / (pn[None, :] * sn + 1e-8)
    aux_ref[...] = jnp.concatenate(
        [nf, ps, jnp.zeros((x.shape[0], 12), jnp.float32)], axis=1)


def _make_aux(features, rsimTrans, pos_embs):
    rows = 2000
    return pl.pallas_call(
        _aux_body,
        grid=(N_NODES // rows,),
        in_specs=[
            pl.BlockSpec((rows, FEAT), lambda i: (i, 0)),
            pl.BlockSpec(rsimTrans.shape, lambda i: (0, 0)),
            pl.BlockSpec((3, CONTRA), lambda i: (0, 0)),
        ],
        out_specs=pl.BlockSpec((rows, 16), lambda i: (i, 0)),
        out_shape=jax.ShapeDtypeStruct((N_NODES, 16), jnp.float32),
    )(features, rsimTrans, pos_embs)


def _sc_body(features, aux, nodes, neigh, tp,
             self_out, agg_out, lsum_out, pos_out,
             nodes_v, nidx_v, tp_v, selfsp, selfaux, gbuf, gaux,
             aggbuf, lsumbuf, pgbuf, posbuf, sem, sem2):
    cid = lax.axis_index("c")
    sid = lax.axis_index("s")
    wid = sid * 2 + cid
    base = wid * BPW
    iota16 = lax.iota(jnp.int32, 16)

    pltpu.sync_copy(nodes.at[pl.ds(base, BPW)], nodes_v)
    for r in range(3):
        pltpu.sync_copy(neigh.at[r, pl.ds(base // 4, BPW // 4)], nidx_v.at[r])
    pltpu.sync_copy(tp.at[pl.ds(wid * TPW, TPW)], tp_v)
    pltpu.async_copy(features.at[nodes_v], selfsp, sem).wait()
    pltpu.async_copy(aux.at[nodes_v], selfaux, sem).wait()
    pltpu.async_copy(aux.at[tp_v], pgbuf, sem).wait()

    nvalid = jnp.clip(P - wid * TPW, 0, TPW)
    vmask = iota16 < nvalid
    posvec = jnp.zeros((16,), jnp.float32)
    for r in range(3):
        col = jnp.full((16,), 1 + r, jnp.int32)
        pvals = plsc.load_gather(pgbuf, [iota16, col])
        dd = jnp.where(vmask, 1.0 - pvals, 0.0)
        posvec = jnp.where(iota16 == r, jnp.sum(dd * dd), posvec)
    posbuf[...] = posvec
    pltpu.sync_copy(posbuf, pos_out.at[wid])

    zcol = jnp.zeros((16,), jnp.int32)

    def idx_of(i, r):
        return nidx_v.at[r, lax.div(i, 4), pl.ds(32 * lax.rem(i, 4), 32)]

    def fire(i, slot):
        for r in range(3):
            idxr = idx_of(i, r)
            pltpu.async_copy(features.at[idxr], gbuf.at[slot, r], sem)
            pltpu.async_copy(aux.at[idxr], gaux.at[slot, r], sem2)

    fire(0, 0)

    def one_center(i, slot, last):
        for r in range(3):
            idxr = idx_of(i, r)
            pltpu.make_async_copy(features.at[idxr], gbuf.at[slot, r],
                                  sem).wait()
            pltpu.make_async_copy(aux.at[idxr], gaux.at[slot, r],
                                  sem2).wait()

        if last:
            @pl.when(i + 1 < BPW)
            def _():
                fire(i + 1, 1 - slot)
        else:
            fire(i + 1, 1 - slot)

        sv = [selfsp[i, pl.ds(16 * j, 16)] for j in range(8)]
        auxrow = selfaux[i, pl.ds(0, 16)]
        nf_self = auxrow[0]
        slotcol = jnp.full((16,), slot, jnp.int32)

        def rel_body(r, lrow):
            dv0 = jnp.zeros((16,), jnp.float32)
            dv1 = jnp.zeros((16,), jnp.float32)
            for k in range(DEG):
                acc = gbuf[slot, r, k, pl.ds(0, 16)] * sv[0]
                for j in range(1, 8):
                    acc = acc + gbuf[slot, r, k, pl.ds(16 * j, 16)] * sv[j]
                tot = jnp.sum(acc)
                if k < 16:
                    dv0 = jnp.where(iota16 == k, tot, dv0)
                else:
                    dv1 = jnp.where(iota16 == (k - 16), tot, dv1)
            rcol = jnp.full((16,), 0, jnp.int32) + r
            nfv0 = plsc.load_gather(gaux, [slotcol, rcol, iota16, zcol])
            nfv1 = plsc.load_gather(gaux, [slotcol, rcol, iota16 + 16, zcol])
            colr = jnp.full((16,), 1, jnp.int32) + r
            ps0 = plsc.load_gather(gaux, [slotcol, rcol, iota16, colr])
            ps1 = plsc.load_gather(gaux, [slotcol, rcol, iota16 + 16, colr])
            s0 = dv0 / (nf_self * nfv0 + 1e-8)
            s1 = dv1 / (nf_self * nfv1 + 1e-8)
            sA, _ = plsc.sort_key_val(s0, s0)
            sB, _ = plsc.sort_key_val(s1, s1)
            tops = jnp.maximum(sA, lax.rev(sB, (0,)))
            t = jnp.min(tops)
            gt0 = s0 > t
            gt1 = s1 > t
            ngt = (plsc.all_reduce_population_count(gt0)[0]
                   + plsc.all_reduce_population_count(gt1)[0])
            need = SAMPLE - ngt
            eq0 = s0 == t
            eq1 = s1 == t
            c0 = plsc.cumsum(eq0.astype(jnp.int32))
            n0 = plsc.all_reduce_population_count(eq0)[0]
            sel0 = eq0 & (c0 <= need)
            c1 = plsc.cumsum(eq1.astype(jnp.int32)) + n0
            sel1 = eq1 & (c1 <= need)
            w0 = jnp.where(gt0 | sel0, 1.0, 0.0)
            w1 = jnp.where(gt1 | sel1, 1.0, 0.0)
            cs = jnp.sum(jnp.where(iota16 == 1 + r, auxrow, 0.0))
            e0 = ps0 - cs
            e1 = ps1 - cs
            lrow = jnp.where(iota16 == r,
                             jnp.sum(w0 * e0 * e0 + w1 * e1 * e1), lrow)
            aj = [jnp.zeros((16,), jnp.float32) for _ in range(8)]
            for k in range(DEG):
                wk = w0[k] if k < 16 else w1[k - 16]
                wsp = lax.broadcast(wk, (16,))
                for j in range(8):
                    aj[j] = aj[j] + wsp * gbuf[slot, r, k, pl.ds(16 * j, 16)]
            arow = jnp.full((16,), 0, jnp.int32) + (r * BPW + i)
            for j in range(8):
                plsc.store_scatter(aggbuf, [arow, iota16 + 16 * j],
                                   aj[j] * (1.0 / SAMPLE))
            return lrow

        lrow = lax.fori_loop(0, 3, rel_body, jnp.zeros((16,), jnp.float32))
        lsumbuf[i] = lrow

    def pair_body(p, carry):
        one_center(2 * p, 0, last=False)
        one_center(2 * p + 1, 1, last=True)
        return carry

    lax.fori_loop(0, BPW // 2, pair_body, 0)

    pltpu.sync_copy(selfsp, self_out.at[pl.ds(base, BPW)])
    for r in range(3):
        pltpu.sync_copy(aggbuf.at[pl.ds(r * BPW, BPW)],
                        agg_out.at[r, pl.ds(base, BPW)])
    pltpu.sync_copy(lsumbuf, lsum_out.at[pl.ds(base, BPW)])


_sc_gather_agg = functools.partial(
    pl.kernel,
    out_type=[
        jax.ShapeDtypeStruct((B, FEAT), jnp.float32),
        jax.ShapeDtypeStruct((3, B, FEAT), jnp.float32),
        jax.ShapeDtypeStruct((B, 16), jnp.float32),
        jax.ShapeDtypeStruct((NW, 16), jnp.float32),
    ],
    mesh=plsc.VectorSubcoreMesh(core_axis_name="c", subcore_axis_name="s",
                                num_cores=2, num_subcores=16),
    compiler_params=pltpu.CompilerParams(needs_layout_passes=False,
                                         use_tc_tiling_on_sc=False),
    scratch_types=[
        pltpu.VMEM((BPW,), jnp.int32),
        pltpu.VMEM((3, BPW // 4, FEAT), jnp.int32),
        pltpu.VMEM((TPW,), jnp.int32),
        pltpu.VMEM((BPW, FEAT), jnp.float32),
        pltpu.VMEM((BPW, 16), jnp.float32),
        pltpu.VMEM((2, 3, DEG, FEAT), jnp.float32),
        pltpu.VMEM((2, 3, DEG, 16), jnp.float32),
        pltpu.VMEM((3 * BPW, FEAT), jnp.float32),
        pltpu.VMEM((BPW, 16), jnp.float32),
        pltpu.VMEM((TPW, 16), jnp.float32),
        pltpu.VMEM((16,), jnp.float32),
        pltpu.SemaphoreType.DMA,
        pltpu.SemaphoreType.DMA,
    ],
)(_sc_body)


def _comb_body(self_ref, a1_ref, a2_ref, a3_ref, w1_ref, w2_ref, w3_ref,
               wt_ref, lsum_ref, pos_ref, out_ref, loss_ref):
    sf = self_ref[...]
    r1 = jnp.maximum(jnp.dot(a1_ref[0], w1_ref[...],
                             preferred_element_type=jnp.float32), 0.0)
    r2 = jnp.maximum(jnp.dot(a2_ref[0], w2_ref[...],
                             preferred_element_type=jnp.float32), 0.0)
    r3 = jnp.maximum(jnp.dot(a3_ref[0], w3_ref[...],
                             preferred_element_type=jnp.float32), 0.0)
    cat = jnp.concatenate([sf, r1, r2, r3], axis=1)
    out = lax.dot_general(wt_ref[...], cat, (((0,), (1,)), ((), ())),
                          preferred_element_type=jnp.float32)
    out_ref[...] = jnp.maximum(out, 0.0)
    mask = lax.broadcasted_iota(jnp.int32, (1, 16), 1) < 3
    part = jnp.sum(jnp.where(mask, lsum_ref[...], 0.0)) / (B * SAMPLE * 3)

    @pl.when(pl.program_id(0) == 0)
    def _():
        pt = jnp.sum(jnp.where(mask, pos_ref[...], 0.0)) / (P * 3)
        loss_ref[...] = jnp.full((1, 1), 0.0) + pt

    loss_ref[...] = loss_ref[...] + part


def _combine(selfb, agg, lsum, posp, W1, W2, W3, weight):
    blk = 1024
    aspec = lambda r: pl.BlockSpec((1, blk, FEAT), lambda i, _r=r: (_r, i, 0))
    return pl.pallas_call(
        _comb_body,
        grid=(B // blk,),
        in_specs=[
            pl.BlockSpec((blk, FEAT), lambda i: (i, 0)),
            aspec(0), aspec(1), aspec(2),
            pl.BlockSpec((FEAT, EMBED), lambda i: (0, 0)),
            pl.BlockSpec((FEAT, EMBED), lambda i: (0, 0)),
            pl.BlockSpec((FEAT, EMBED), lambda i: (0, 0)),
            pl.BlockSpec((FEAT + 3 * EMBED, EMBED), lambda i: (0, 0)),
            pl.BlockSpec((blk, 16), lambda i: (i, 0)),
            pl.BlockSpec((NW, 16), lambda i: (0, 0)),
        ],
        out_specs=[
            pl.BlockSpec((EMBED, blk), lambda i: (0, i)),
            pl.BlockSpec((1, 1), lambda i: (0, 0)),
        ],
        out_shape=[
            jax.ShapeDtypeStruct((EMBED, B), jnp.float32),
            jax.ShapeDtypeStruct((1, 1), jnp.float32),
        ],
    )(selfb, agg, agg, agg, W1, W2, W3, weight, lsum, posp)


def kernel(nodes, labels, neigh1, neigh2, neigh3, train_pos, features,
           weight, rsimTrans, pos_embs, W1, W2, W3):
    nodes = nodes.astype(jnp.int32)
    neigh = jnp.stack([neigh1, neigh2, neigh3]).astype(jnp.int32)
    neigh = neigh.reshape(3, B // 4, 4 * DEG)
    tp = jnp.pad(train_pos.astype(jnp.int32), (0, TP_PAD - P))
    aux = _make_aux(features, rsimTrans, pos_embs)
    selfb, agg, lsum, posp = _sc_gather_agg(features, aux, nodes, neigh, tp)
    combined, loss = _combine(selfb, agg, lsum, posp, W1, W2, W3, weight)
    return combined, loss.reshape(())
